# baseline (speedup 1.0000x reference)
import jax
import jax.numpy as jnp
from jax import lax
from jax.experimental import pallas as pl
from jax.experimental.pallas import tpu as pltpu
from jax.experimental.pallas import tpu_sc as plsc
import functools

N_EDGES = 1600000
EMB = 32
NC, NS = 2, 16
NW = NC * NS
E_PER_W = N_EDGES // NW
CHUNK = 2000
NCHUNK = E_PER_W // CHUNK
PIECE = 80
NPIECE = CHUNK // PIECE


def _body(ef_hbm, ctab_hbm, out_hbm, ef_v, c2d, rows, sem):
    wid = lax.axis_index("s") * NC + lax.axis_index("c")

    def chunk_body(g, carry):
        eoff = wid * E_PER_W + g * CHUNK
        pltpu.sync_copy(ef_hbm.at[pl.ds(eoff * 3, CHUNK * 3)], ef_v)

        def t_body(t, carry2):
            iota = lax.broadcasted_iota(jnp.int32, (16,), 0)
            v0 = ef_v[pl.ds(t * 48, 16)]
            v1 = ef_v[pl.ds(t * 48 + 16, 16)]
            v2 = ef_v[pl.ds(t * 48 + 32, 16)]

            dnums = lax.GatherDimensionNumbers(
                offset_dims=(), collapsed_slice_dims=(0,), start_index_map=(0,)
            )

            def vgather(v, lane):
                return lax.gather(
                    v,
                    lane[:, None],
                    dnums,
                    (1,),
                    mode=lax.GatherScatterMode.PROMISE_IN_BOUNDS,
                )

            def field(f):
                w = iota * 3 + f
                lane = lax.rem(w, 16)
                j = lax.div(w, 16)
                return jnp.where(
                    j == 0,
                    vgather(v0, lane),
                    jnp.where(j == 1, vgather(v1, lane), vgather(v2, lane)),
                )

            c = field(0) * 12 + field(1) * 2 + field(2)
            c2d[pl.ds(t * 16, 16)] = c
            return carry2

        lax.fori_loop(0, CHUNK // 16, t_body, 0)

        copies = [
            pltpu.async_copy(
                ctab_hbm.at[c2d.at[pl.ds(p * PIECE, PIECE)]],
                rows.at[pl.ds(p * PIECE, PIECE)],
                sem,
            )
            for p in range(NPIECE)
        ]
        for cp in copies:
            cp.wait()

        pltpu.sync_copy(rows, out_hbm.at[pl.ds(eoff, CHUNK)])
        return carry

    lax.fori_loop(0, NCHUNK, chunk_body, 0)


@jax.jit
def kernel(edge_feature, W0, W1, W2):
    ef = edge_feature.astype(jnp.int32).reshape(-1)
    ctab = (
        W0[:, None, None, :] + W1[None, :, None, :] + W2[None, None, :, :]
    ).reshape(60, EMB)

    run = pl.kernel(
        _body,
        out_type=jax.ShapeDtypeStruct((N_EDGES, EMB), jnp.float32),
        mesh=plsc.VectorSubcoreMesh(core_axis_name="c", subcore_axis_name="s"),
        compiler_params=pltpu.CompilerParams(use_tc_tiling_on_sc=False),
        scratch_types=[
            pltpu.VMEM((CHUNK * 3,), jnp.int32),
            pltpu.VMEM((CHUNK,), jnp.int32),
            pltpu.VMEM((CHUNK, EMB), jnp.float32),
            pltpu.SemaphoreType.DMA,
        ],
    )
    return run(ef, ctab)


# speedup vs baseline: 1.7464x; 1.7464x over previous
import jax
import jax.numpy as jnp
from jax import lax
from jax.experimental import pallas as pl
from jax.experimental.pallas import tpu as pltpu
from jax.experimental.pallas import tpu_sc as plsc

N_EDGES = 1600000
EMB = 32
NC, NS = 2, 16
NW = NC * NS
E_PER_W = N_EDGES // NW
CHUNK = 400
NCHUNK = E_PER_W // CHUNK
NT = CHUNK // 16


def _body(ef_hbm, ctab_hbm, out_hbm, ctab_v, ef_v, rows, sem):
    wid = lax.axis_index("s") * NC + lax.axis_index("c")
    pltpu.sync_copy(ctab_hbm, ctab_v)

    def chunk_body(g, carry):
        eoff = wid * E_PER_W + g * CHUNK
        pltpu.sync_copy(ef_hbm.at[pl.ds(eoff * 3, CHUNK * 3)], ef_v)

        def t_body(t, carry2):
            iota = lax.broadcasted_iota(jnp.int32, (16,), 0)
            idx = t * 48 + iota * 3
            i0 = plsc.load_gather(ef_v, [idx])
            i1 = plsc.load_gather(ef_v, [idx + 1])
            i2 = plsc.load_gather(ef_v, [idx + 2])
            c32 = (i0 * 12 + i1 * 2 + i2) * EMB
            obase = t * (16 * EMB) + iota * EMB
            for col in range(EMB):
                v = plsc.load_gather(ctab_v, [c32 + col])
                plsc.store_scatter(rows, [obase + col], v)
            return carry2

        lax.fori_loop(0, NT, t_body, 0)

        pltpu.sync_copy(rows, out_hbm.at[pl.ds(eoff * EMB, CHUNK * EMB)])
        return carry

    lax.fori_loop(0, NCHUNK, chunk_body, 0)


@jax.jit
def kernel(edge_feature, W0, W1, W2):
    ef = edge_feature.astype(jnp.int32).reshape(-1)
    ctab = (
        W0[:, None, None, :] + W1[None, :, None, :] + W2[None, None, :, :]
    ).reshape(-1)

    run = pl.kernel(
        _body,
        out_type=jax.ShapeDtypeStruct((N_EDGES * EMB,), jnp.float32),
        mesh=plsc.VectorSubcoreMesh(core_axis_name="c", subcore_axis_name="s"),
        compiler_params=pltpu.CompilerParams(
            use_tc_tiling_on_sc=False, needs_layout_passes=False
        ),
        scratch_types=[
            pltpu.VMEM((60 * EMB,), jnp.float32),
            pltpu.VMEM((CHUNK * 3,), jnp.int32),
            pltpu.VMEM((CHUNK * EMB,), jnp.float32),
            pltpu.SemaphoreType.DMA,
        ],
    )
    return run(ef, ctab).reshape(N_EDGES, EMB)


# speedup vs baseline: 8.0007x; 4.5812x over previous
import jax
import jax.numpy as jnp
from jax import lax
from jax.experimental import pallas as pl
from jax.experimental.pallas import tpu as pltpu
from jax.experimental.pallas import tpu_sc as plsc

N_EDGES = 1600000
EMB = 32
NC, NS = 2, 16
NW = NC * NS
NBLK = N_EDGES // 128
BLK_PER_W = NBLK // NW
CB = 10
NCHUNK = BLK_PER_W // CB


def _body(ef_hbm, ctab_hbm, out_hbm, ctab_v, ef_v, rows, sem):
    wid = lax.axis_index("s") * NC + lax.axis_index("c")
    pltpu.sync_copy(ctab_hbm, ctab_v)

    def do_blocks(blk0, nb):
        for f in range(3):
            pltpu.sync_copy(
                ef_hbm.at[pl.ds(f * N_EDGES + blk0 * 128, nb * 128)],
                ef_v.at[pl.ds(f * (CB * 128), nb * 128)],
            )

        def ebl_body(ebl, carry):
            def grp_body(grp, carry2):
                el0 = ebl * 128 + grp * 16
                i0 = ef_v[pl.ds(el0, 16)]
                i1 = ef_v[pl.ds(CB * 128 + el0, 16)]
                i2 = ef_v[pl.ds(2 * CB * 128 + el0, 16)]
                cm = (i0 * 12 + i1 * 2 + i2) * EMB
                for c in range(EMB):
                    v = plsc.load_gather(ctab_v, [cm + c])
                    rows[pl.ds((c // 8) * (nb * 1024) + ebl * 1024
                               + (c % 8) * 128 + grp * 16, 16)] = v
                return carry2

            return lax.fori_loop(0, 8, grp_body, carry)

        lax.fori_loop(0, nb, ebl_body, 0)

        for cb in range(4):
            pltpu.sync_copy(
                rows.at[pl.ds(cb * (nb * 1024), nb * 1024)],
                out_hbm.at[pl.ds((cb * NBLK + blk0) * 1024, nb * 1024)],
            )

    def chunk_body(g, carry):
        do_blocks(wid * BLK_PER_W + g * CB, CB)
        return carry

    lax.fori_loop(0, NCHUNK, chunk_body, 0)

    @pl.when(wid < 20)
    def _():
        do_blocks(NW * BLK_PER_W + wid, 1)


@jax.jit
def kernel(edge_feature, W0, W1, W2):
    eft = edge_feature.astype(jnp.int32).T.reshape(-1)
    ctab = (
        W0[:, None, None, :] + W1[None, :, None, :] + W2[None, None, :, :]
    ).reshape(-1)

    run = pl.kernel(
        _body,
        out_type=jax.ShapeDtypeStruct((N_EDGES * EMB,), jnp.float32),
        mesh=plsc.VectorSubcoreMesh(core_axis_name="c", subcore_axis_name="s"),
        compiler_params=pltpu.CompilerParams(
            use_tc_tiling_on_sc=False, needs_layout_passes=False
        ),
        scratch_types=[
            pltpu.VMEM((60 * EMB,), jnp.float32),
            pltpu.VMEM((3 * CB * 128,), jnp.int32),
            pltpu.VMEM((4 * CB * 1024,), jnp.float32),
            pltpu.SemaphoreType.DMA,
        ],
    )
    out = run(eft, ctab)
    return out.reshape(4, NBLK, 8, 128).transpose(1, 3, 0, 2).reshape(N_EDGES, EMB)


# speedup vs baseline: 11.7755x; 1.4718x over previous
import jax
import jax.numpy as jnp
from jax import lax
from jax.experimental import pallas as pl
from jax.experimental.pallas import tpu as pltpu
from jax.experimental.pallas import tpu_sc as plsc

N_EDGES = 1600000
EMB = 32
NC, NS = 2, 16
NW = NC * NS
NBLK = N_EDGES // 128
BLK_PER_W = NBLK // NW
CB = 10
NCHUNK = BLK_PER_W // CB


def _body(ef_hbm, ctab_hbm, out_hbm, ctab_v, ef_v, rows, sem):
    wid = lax.axis_index("s") * NC + lax.axis_index("c")
    pltpu.sync_copy(ctab_hbm, ctab_v)

    def do_blocks(blk0, nb):
        for f in range(3):
            pltpu.sync_copy(
                ef_hbm.at[pl.ds(f * N_EDGES + blk0 * 128, nb * 128)],
                ef_v.at[pl.ds(f * (CB * 128), nb * 128)],
            )

        @plsc.parallel_loop(0, nb * 8, unroll=2)
        def _(g2):
            el0 = g2 * 16
            i0 = ef_v[pl.ds(el0, 16)]
            i1 = ef_v[pl.ds(CB * 128 + el0, 16)]
            i2 = ef_v[pl.ds(2 * CB * 128 + el0, 16)]
            cm = (i0 * 12 + i1 * 2 + i2) * EMB
            ebl = lax.div(g2, 8)
            base = ebl * 1024 + (g2 - ebl * 8) * 16
            vs = [plsc.load_gather(ctab_v, [cm + c]) for c in range(EMB)]
            for c in range(EMB):
                rows[pl.ds((c // 8) * (nb * 1024) + base + (c % 8) * 128, 16)] = vs[c]

        for cb in range(4):
            pltpu.sync_copy(
                rows.at[pl.ds(cb * (nb * 1024), nb * 1024)],
                out_hbm.at[pl.ds((cb * NBLK + blk0) * 1024, nb * 1024)],
            )

    def chunk_body(g, carry):
        do_blocks(wid * BLK_PER_W + g * CB, CB)
        return carry

    lax.fori_loop(0, NCHUNK, chunk_body, 0)

    @pl.when(wid < 20)
    def _():
        do_blocks(NW * BLK_PER_W + wid, 1)


@jax.jit
def kernel(edge_feature, W0, W1, W2):
    eft = edge_feature.astype(jnp.int32).T.reshape(-1)
    ctab = (
        W0[:, None, None, :] + W1[None, :, None, :] + W2[None, None, :, :]
    ).reshape(-1)

    run = pl.kernel(
        _body,
        out_type=jax.ShapeDtypeStruct((N_EDGES * EMB,), jnp.float32),
        mesh=plsc.VectorSubcoreMesh(core_axis_name="c", subcore_axis_name="s"),
        compiler_params=pltpu.CompilerParams(
            use_tc_tiling_on_sc=False, needs_layout_passes=False
        ),
        scratch_types=[
            pltpu.VMEM((60 * EMB,), jnp.float32),
            pltpu.VMEM((3 * CB * 128,), jnp.int32),
            pltpu.VMEM((4 * CB * 1024,), jnp.float32),
            pltpu.SemaphoreType.DMA,
        ],
    )
    out = run(eft, ctab)
    return out.reshape(4, NBLK, 8, 128).transpose(1, 3, 0, 2).reshape(N_EDGES, EMB)
